# baseline (speedup 1.0000x reference)
import jax
import jax.numpy as jnp
from jax.experimental import pallas as pl
from jax.experimental.pallas import tpu as pltpu

B = 16
WIN = 64
N = 32
D = 64
HG = 64
QK = 32
HOR = 8
BN = B * N
KTOT = WIN * N * D
KBLK = 8192
NBUF = 2
NCHUNK = KTOT // KBLK
TPC = KBLK // (N * D)


def _mm(a, b):
    return jax.lax.dot_general(a, b, (((1,), (0,)), ((), ())),
                               preferred_element_type=jnp.float32)


def _mmT(a, b):
    return jax.lax.dot_general(a, b, (((1,), (1,)), ((), ())),
                               preferred_element_type=jnp.float32)


def _conv_block_pieces(m, taps, cb, gnext):
    k = len(taps)
    pad = k // 2
    pieces = [m[:, t * D:(t + 1) * D] for t in range(WIN)]
    outs = []
    for t in range(WIN):
        acc = None
        for u in range(k):
            tt = t + u - pad
            if 0 <= tt < WIN:
                q = _mm(pieces[tt], taps[u])
                acc = q if acc is None else acc + q
        r = acc + cb
        r = jnp.where(r > 0.0, r, 0.01 * r)
        outs.append(_mm(r, gnext) if gnext is not None else r)
    return outs


def _mega_kernel(xg_ref, wr_ref, wz_ref, wn_ref,
                 whr_ref, whz_ref, whn_ref,
                 bir_ref, biz_ref, bin_ref,
                 bhr_ref, bhz_ref, bhn_ref,
                 wq_ref, wqb_ref, wk_ref, wkb_ref,
                 cw_ref, cb_ref,
                 g0_ref, gb0_ref, t00_ref, t01_ref, t02_ref, cb0_ref,
                 g1_ref, gb1_ref, t10_ref, t11_ref, t12_ref, t13_ref,
                 t14_ref, cb1_ref, w_ref, lb_ref, o_ref, wbuf, wsem):
    def wcopy(i):
        return pltpu.make_async_copy(
            w_ref.at[:, pl.ds(i * KBLK, KBLK)],
            wbuf.at[i % NBUF], wsem.at[i % NBUF])

    for i in range(NBUF):
        wcopy(i).start()

    wr = wr_ref[...]
    wz = wz_ref[...]
    wn = wn_ref[...]
    whr = whr_ref[...]
    whz = whz_ref[...]
    whn = whn_ref[...]
    bir = bir_ref[...]
    biz = biz_ref[...]
    bin_ = bin_ref[...]
    bhr = bhr_ref[...]
    bhz = bhz_ref[...]
    bhn = bhn_ref[...]

    def step(t, h):
        xt = xg_ref[t]
        r = jax.nn.sigmoid(xt * wr + bir + _mm(h, whr) + bhr)
        z = jax.nn.sigmoid(xt * wz + biz + _mm(h, whz) + bhz)
        n = jnp.tanh(xt * wn + bin_ + r * (_mm(h, whn) + bhn))
        return (1.0 - z) * n + z * h

    h = jax.lax.fori_loop(0, WIN, step, jnp.zeros((BN, HG), jnp.float32))

    q = _mm(h, wq_ref[...]) + wqb_ref[...]
    k = _mm(h, wk_ref[...]) + wkb_ref[...]
    s = _mmT(q, k) * (1.0 / (QK ** 0.5))
    rid = jax.lax.broadcasted_iota(jnp.int32, (BN, BN), 0) % B
    cid = jax.lax.broadcasted_iota(jnp.int32, (BN, BN), 1) % B
    s = jnp.where(rid == cid, s, -1e30)
    m = jnp.max(s, axis=1, keepdims=True)
    e = jnp.exp(s - m)
    wm = e / jnp.sum(e, axis=1, keepdims=True)
    wt = wm.T
    deg_c = jnp.sum(wt, axis=1, keepdims=True)
    deg_l = jnp.sum(wm, axis=0, keepdims=True)
    dc = jnp.where(deg_c > 0, jax.lax.rsqrt(deg_c), 0.0)
    dl = jnp.where(deg_l > 0, jax.lax.rsqrt(deg_l), 0.0)
    at_bd = dc * wt * dl

    cw = cw_ref[...]
    cb = cb_ref[...]
    g0 = g0_ref[...]
    h0_pieces = []
    for t in range(WIN):
        nt = xg_ref[t] * cw + cb
        h0_pieces.append(_mm(nt, g0))
    m0 = _mm(at_bd, jnp.concatenate(h0_pieces, axis=1)) + gb0_ref[...]
    h1 = _conv_block_pieces(
        m0, (t00_ref[...], t01_ref[...], t02_ref[...]), cb0_ref[...],
        g1_ref[...])
    m1 = _mm(at_bd, jnp.concatenate(h1, axis=1)) + gb1_ref[...]
    outs = _conv_block_pieces(
        m1, (t10_ref[...], t11_ref[...], t12_ref[...], t13_ref[...],
             t14_ref[...]), cb1_ref[...], None)
    chunks = []
    for t in range(WIN):
        p3 = jnp.transpose(outs[t].reshape(N, B, D), (0, 2, 1))
        chunks.append(p3.reshape(N * D, B).T)

    acc = jnp.broadcast_to(lb_ref[...], (B, N * HOR))
    for i in range(NCHUNK):
        wcopy(i).wait()
        xi = jnp.concatenate(chunks[TPC * i:TPC * (i + 1)], axis=1)
        acc = acc + _mmT(xi, wbuf[i % NBUF])
        if i + NBUF < NCHUNK:
            wcopy(i + NBUF).start()
    o_ref[...] = acc


def kernel(x, gru_W_ih, gru_W_hh, gru_b_ih, gru_b_hh, wq_W, wq_b, wk_W, wk_b,
           conv_in_w, conv_in_b, gcn_w0, gcn_b0, conv_w0, conv_b0,
           gcn_w1, gcn_b1, conv_w1, conv_b1, lout_W, lout_b):
    f32 = jnp.float32
    xg = jnp.transpose(x, (1, 2, 0)).reshape(WIN, BN, 1)

    wih = gru_W_ih.reshape(3 * HG)
    wr, wz, wn = (wih[i * HG:(i + 1) * HG][None, :] for i in range(3))
    whr, whz, whn = (gru_W_hh[i * HG:(i + 1) * HG].T for i in range(3))
    bir, biz, bin_ = (gru_b_ih[i * HG:(i + 1) * HG][None, :] for i in range(3))
    bhr, bhz, bhn = (gru_b_hh[i * HG:(i + 1) * HG][None, :] for i in range(3))
    taps0 = [conv_w0[:, :, u].T for u in range(3)]
    taps1 = [conv_w1[:, :, u].T for u in range(5)]

    n_in = 35
    specs = [pl.BlockSpec(memory_space=pltpu.MemorySpace.VMEM)] * n_in
    specs[33] = pl.BlockSpec(memory_space=pltpu.MemorySpace.HBM)

    out = pl.pallas_call(
        _mega_kernel,
        in_specs=specs,
        out_specs=pl.BlockSpec(memory_space=pltpu.MemorySpace.VMEM),
        out_shape=jax.ShapeDtypeStruct((B, N * HOR), f32),
        scratch_shapes=[
            pltpu.VMEM((NBUF, N * HOR, KBLK), f32),
            pltpu.SemaphoreType.DMA((NBUF,)),
        ],
    )(xg, wr, wz, wn, whr, whz, whn, bir, biz, bin_, bhr, bhz, bhn,
      wq_W.T, wq_b[None, :], wk_W.T, wk_b[None, :],
      conv_in_w.reshape(1, D), conv_in_b[None, :],
      gcn_w0.T, jnp.tile(gcn_b0, WIN)[None, :], *taps0, conv_b0[None, :],
      gcn_w1.T, jnp.tile(gcn_b1, WIN)[None, :], *taps1, conv_b1[None, :],
      lout_W, lout_b[None, :])

    return out.reshape(B, HOR, N)


# speedup vs baseline: 1.0637x; 1.0637x over previous
import jax
import jax.numpy as jnp
from jax.experimental import pallas as pl
from jax.experimental.pallas import tpu as pltpu

B = 16
WIN = 64
N = 32
D = 64
HG = 64
QK = 32
HOR = 8
BN = B * N
KTOT = WIN * N * D
KBLK = 4096
NBUF = 5
NCHUNK = KTOT // KBLK
TPC = KBLK // (N * D)


def _mm(a, b):
    return jax.lax.dot_general(a, b, (((1,), (0,)), ((), ())),
                               preferred_element_type=jnp.float32)


def _mmT(a, b):
    return jax.lax.dot_general(a, b, (((1,), (1,)), ((), ())),
                               preferred_element_type=jnp.float32)


def _conv_block_pieces(m, taps, cb, gnext):
    k = len(taps)
    pad = k // 2
    pieces = [m[:, t * D:(t + 1) * D] for t in range(WIN)]
    outs = []
    for t in range(WIN):
        acc = None
        for u in range(k):
            tt = t + u - pad
            if 0 <= tt < WIN:
                q = _mm(pieces[tt], taps[u])
                acc = q if acc is None else acc + q
        r = acc + cb
        r = jnp.where(r > 0.0, r, 0.01 * r)
        outs.append(_mm(r, gnext) if gnext is not None else r)
    return outs


def _mega_kernel(xg_ref, wr_ref, wz_ref, wn_ref,
                 whr_ref, whz_ref, whn_ref,
                 bir_ref, biz_ref, bin_ref,
                 bhr_ref, bhz_ref, bhn_ref,
                 wq_ref, wqb_ref, wk_ref, wkb_ref,
                 cw_ref, cb_ref,
                 g0_ref, gb0_ref, t00_ref, t01_ref, t02_ref, cb0_ref,
                 g1_ref, gb1_ref, t10_ref, t11_ref, t12_ref, t13_ref,
                 t14_ref, cb1_ref, w_ref, lb_ref, o_ref, wbuf, wsem):
    def wcopy(i):
        return pltpu.make_async_copy(
            w_ref.at[:, pl.ds(i * KBLK, KBLK)],
            wbuf.at[i % NBUF], wsem.at[i % NBUF])

    for i in range(NBUF):
        wcopy(i).start()

    wr = wr_ref[...]
    wz = wz_ref[...]
    wn = wn_ref[...]
    whr = whr_ref[...]
    whz = whz_ref[...]
    whn = whn_ref[...]
    bir = bir_ref[...]
    biz = biz_ref[...]
    bin_ = bin_ref[...]
    bhr = bhr_ref[...]
    bhz = bhz_ref[...]
    bhn = bhn_ref[...]

    def step(t, h):
        xt = xg_ref[t]
        r = jax.nn.sigmoid(xt * wr + bir + _mm(h, whr) + bhr)
        z = jax.nn.sigmoid(xt * wz + biz + _mm(h, whz) + bhz)
        n = jnp.tanh(xt * wn + bin_ + r * (_mm(h, whn) + bhn))
        return (1.0 - z) * n + z * h

    h = jax.lax.fori_loop(0, WIN, step, jnp.zeros((BN, HG), jnp.float32))

    q = _mm(h, wq_ref[...]) + wqb_ref[...]
    k = _mm(h, wk_ref[...]) + wkb_ref[...]
    s = _mmT(q, k) * (1.0 / (QK ** 0.5))
    rid = jax.lax.broadcasted_iota(jnp.int32, (BN, BN), 0) % B
    cid = jax.lax.broadcasted_iota(jnp.int32, (BN, BN), 1) % B
    s = jnp.where(rid == cid, s, -1e30)
    m = jnp.max(s, axis=1, keepdims=True)
    e = jnp.exp(s - m)
    wm = e / jnp.sum(e, axis=1, keepdims=True)
    wt = wm.T
    deg_c = jnp.sum(wt, axis=1, keepdims=True)
    deg_l = jnp.sum(wm, axis=0, keepdims=True)
    dc = jnp.where(deg_c > 0, jax.lax.rsqrt(deg_c), 0.0)
    dl = jnp.where(deg_l > 0, jax.lax.rsqrt(deg_l), 0.0)
    at_bd = dc * wt * dl

    cw = cw_ref[...]
    cb = cb_ref[...]
    g0 = g0_ref[...]
    h0_pieces = []
    for t in range(WIN):
        nt = xg_ref[t] * cw + cb
        h0_pieces.append(_mm(nt, g0))
    m0 = _mm(at_bd, jnp.concatenate(h0_pieces, axis=1)) + gb0_ref[...]
    h1 = _conv_block_pieces(
        m0, (t00_ref[...], t01_ref[...], t02_ref[...]), cb0_ref[...],
        g1_ref[...])
    m1 = _mm(at_bd, jnp.concatenate(h1, axis=1)) + gb1_ref[...]
    outs = _conv_block_pieces(
        m1, (t10_ref[...], t11_ref[...], t12_ref[...], t13_ref[...],
             t14_ref[...]), cb1_ref[...], None)
    chunks = []
    for t in range(WIN):
        p3 = jnp.transpose(outs[t].reshape(N, B, D), (0, 2, 1))
        chunks.append(p3.reshape(N * D, B).T)

    acc = jnp.broadcast_to(lb_ref[...], (B, N * HOR))
    for i in range(NCHUNK):
        wcopy(i).wait()
        xi = jnp.concatenate(chunks[TPC * i:TPC * (i + 1)], axis=1)
        acc = acc + _mmT(xi, wbuf[i % NBUF])
        if i + NBUF < NCHUNK:
            wcopy(i + NBUF).start()
    o_ref[...] = acc


def kernel(x, gru_W_ih, gru_W_hh, gru_b_ih, gru_b_hh, wq_W, wq_b, wk_W, wk_b,
           conv_in_w, conv_in_b, gcn_w0, gcn_b0, conv_w0, conv_b0,
           gcn_w1, gcn_b1, conv_w1, conv_b1, lout_W, lout_b):
    f32 = jnp.float32
    xg = jnp.transpose(x, (1, 2, 0)).reshape(WIN, BN, 1)

    wih = gru_W_ih.reshape(3 * HG)
    wr, wz, wn = (wih[i * HG:(i + 1) * HG][None, :] for i in range(3))
    whr, whz, whn = (gru_W_hh[i * HG:(i + 1) * HG].T for i in range(3))
    bir, biz, bin_ = (gru_b_ih[i * HG:(i + 1) * HG][None, :] for i in range(3))
    bhr, bhz, bhn = (gru_b_hh[i * HG:(i + 1) * HG][None, :] for i in range(3))
    taps0 = [conv_w0[:, :, u].T for u in range(3)]
    taps1 = [conv_w1[:, :, u].T for u in range(5)]

    n_in = 35
    specs = [pl.BlockSpec(memory_space=pltpu.MemorySpace.VMEM)] * n_in
    specs[33] = pl.BlockSpec(memory_space=pltpu.MemorySpace.HBM)

    out = pl.pallas_call(
        _mega_kernel,
        in_specs=specs,
        out_specs=pl.BlockSpec(memory_space=pltpu.MemorySpace.VMEM),
        out_shape=jax.ShapeDtypeStruct((B, N * HOR), f32),
        scratch_shapes=[
            pltpu.VMEM((NBUF, N * HOR, KBLK), f32),
            pltpu.SemaphoreType.DMA((NBUF,)),
        ],
    )(xg, wr, wz, wn, whr, whz, whn, bir, biz, bin_, bhr, bhz, bhn,
      wq_W.T, wq_b[None, :], wk_W.T, wk_b[None, :],
      conv_in_w.reshape(1, D), conv_in_b[None, :],
      gcn_w0.T, jnp.tile(gcn_b0, WIN)[None, :], *taps0, conv_b0[None, :],
      gcn_w1.T, jnp.tile(gcn_b1, WIN)[None, :], *taps1, conv_b1[None, :],
      lout_W, lout_b[None, :])

    return out.reshape(B, HOR, N)
